# baseline (speedup 1.0000x reference)
import functools

import jax
import jax.numpy as jnp
from jax import lax
from jax.experimental import pallas as pl
from jax.experimental.pallas import tpu as pltpu
from jax.experimental.pallas import tpu_sc as plsc

VOCAB = 100000
HIDDEN = 2048

NC = 2
NS = 16
NW = NC * NS

B = 4 * 4096
CHUNK = 16
NCHUNK = 23
B_SC = NW * NCHUNK * CHUNK
B_TC = B - B_SC
G = 64

_mesh = plsc.VectorSubcoreMesh(core_axis_name="c", subcore_axis_name="s")


@functools.partial(
    pl.kernel,
    out_type=jax.ShapeDtypeStruct((B_SC, HIDDEN), jnp.float32),
    mesh=_mesh,
    scratch_types=[
        pltpu.VMEM((NCHUNK, CHUNK), jnp.int32),
        pltpu.VMEM((3, CHUNK, HIDDEN), jnp.float32),
        pltpu.SemaphoreType.DMA,
        pltpu.SemaphoreType.DMA,
    ],
)
def _embed_lookup_sc(ids_hbm, table_hbm, out_hbm, idx_v, rows_v, gsem, ssem):
    wid = lax.axis_index("s") * NC + lax.axis_index("c")
    pltpu.sync_copy(ids_hbm.at[wid], idx_v)
    out_base = wid * NCHUNK

    def gather(c, b):
        pltpu.async_copy(table_hbm.at[idx_v.at[c]], rows_v.at[b], gsem)

    def scatter(c, b):
        row0 = (out_base + c) * CHUNK
        pltpu.async_copy(rows_v.at[b], out_hbm.at[pl.ds(row0, CHUNK)], ssem)

    def wait_gather(b):
        pltpu.make_async_copy(
            table_hbm.at[pl.ds(0, CHUNK)], rows_v.at[b], gsem).wait()

    def wait_scatter(b):
        pltpu.make_async_copy(
            rows_v.at[b], out_hbm.at[pl.ds(0, CHUNK)], ssem).wait()

    gather(0, 0)
    gather(1, 1)
    wait_gather(0)
    gather(2, 2)
    scatter(0, 0)

    @pl.loop(1, NCHUNK - 4, step=3)
    def _(j):
        for b in range(3):
            c = j + b
            wait_gather((1 + b) % 3)
            wait_scatter(b % 3)
            gather(c + 2, b % 3)
            scatter(c, (1 + b) % 3)

    wait_gather(1)
    wait_scatter(0)
    gather(NCHUNK - 2, 0)
    scatter(NCHUNK - 4, 1)

    wait_gather(2)
    wait_scatter(1)
    gather(NCHUNK - 1, 1)
    scatter(NCHUNK - 3, 2)

    wait_gather(0)
    wait_scatter(2)
    scatter(NCHUNK - 2, 0)

    wait_gather(1)
    wait_scatter(0)
    scatter(NCHUNK - 1, 1)
    wait_scatter(1)


def _tc_body(ids_sref, *refs):
    del ids_sref
    in_refs, out_ref = refs[:G], refs[G]
    for g in range(G):
        out_ref[g, :] = in_refs[g][0, 0, :]


def _tc_in_spec(g):
    return pl.BlockSpec((1, 1, HIDDEN), lambda i, ids: (ids[i * G + g], 0, 0))


_tc_gather = pl.pallas_call(
    _tc_body,
    grid_spec=pltpu.PrefetchScalarGridSpec(
        num_scalar_prefetch=1,
        grid=(B_TC // G,),
        in_specs=[_tc_in_spec(g) for g in range(G)],
        out_specs=pl.BlockSpec((G, HIDDEN), lambda i, ids: (i, 0)),
    ),
    out_shape=jax.ShapeDtypeStruct((B_TC, HIDDEN), jnp.float32),
    compiler_params=pltpu.CompilerParams(
        dimension_semantics=("arbitrary",)),
)


def kernel(input_ids, embed_tokens):
    flat = input_ids.reshape(-1)
    ids_sc = flat[:B_SC].reshape(NW, NCHUNK, CHUNK)
    ids_tc = flat[B_SC:]
    out_sc = _embed_lookup_sc(ids_sc, embed_tokens)
    table3 = embed_tokens.reshape(VOCAB, 1, HIDDEN)
    out_tc = _tc_gather(ids_tc, *([table3] * G))
    out = jnp.concatenate([out_sc, out_tc], axis=0)
    return out.reshape(input_ids.shape[0], input_ids.shape[1], HIDDEN)
